# baseline (speedup 1.0000x reference)
import functools

import numpy as np
import jax
import jax.numpy as jnp
from jax import lax
from jax.experimental import pallas as pl
from jax.experimental.pallas import tpu as pltpu
from jax.experimental.pallas import tpu_sc as plsc

LANES = 128
ILANES = 3 * LANES
BR = 256
NW = 32

_K = np.arange(LANES)
_J = np.arange(ILANES)
_E = (_J[None, :] // 3 == _K[:, None]).astype(np.float32)
_U3 = ((_J[:, None] <= _J[None, :]) &
       (_J[:, None] % 3 == _J[None, :] % 3)).astype(np.float32)
_B = np.arange(BR)
_LS = (_B[None, :] < _B[:, None]).astype(np.float32)
_R3 = (_J[None, :] % 3 == np.arange(3)[:, None]).astype(np.float32)

_HI = lax.Precision.HIGHEST


def _dot(a, b):
    return jnp.dot(a, b, preferred_element_type=jnp.float32, precision=_HI)


def _cumsum_body(v_ref, w_ref, e_ref, u3_ref, ls_ref, r3_ref, out_ref,
                 carry_ref):
    b = pl.program_id(0)

    @pl.when(b == 0)
    def _init():
        carry_ref[...] = jnp.zeros((1, 3), jnp.float32)

    w3 = _dot(w_ref[...], e_ref[...])
    x3 = v_ref[...] * w3
    incl = _dot(x3, u3_ref[...])
    srow3 = incl[:, ILANES - 3:ILANES]
    rowcarry = _dot(ls_ref[...], srow3)
    rc3 = rowcarry + carry_ref[...]
    out_ref[...] = incl + _dot(rc3, r3_ref[...])
    carry_ref[...] = carry_ref[...] + (rowcarry + srow3)[BR - 1:BR, :]


def _stage_a(v_int, w2):
    rows = v_int.shape[0]
    nblocks = rows // BR
    const_spec = lambda shape: pl.BlockSpec(shape, lambda b: (0, 0))
    return pl.pallas_call(
        _cumsum_body,
        grid=(nblocks,),
        in_specs=[
            pl.BlockSpec((BR, ILANES), lambda b: (b, 0)),
            pl.BlockSpec((BR, LANES), lambda b: (b, 0)),
            const_spec((LANES, ILANES)),
            const_spec((ILANES, ILANES)),
            const_spec((BR, BR)),
            const_spec((3, ILANES)),
        ],
        out_specs=pl.BlockSpec((BR, ILANES), lambda b: (b, 0)),
        out_shape=jax.ShapeDtypeStruct((rows, ILANES), jnp.float32),
        scratch_shapes=[pltpu.VMEM((1, 3), jnp.float32)],
    )(v_int, w2, jnp.asarray(_E), jnp.asarray(_U3), jnp.asarray(_LS),
      jnp.asarray(_R3))


def _make_stage_b(n_rays, chunk):
    rpw = n_rays // NW
    mesh = plsc.VectorSubcoreMesh(core_axis_name="c", subcore_axis_name="s")

    @functools.partial(
        pl.kernel,
        mesh=mesh,
        out_type=jax.ShapeDtypeStruct((3 * n_rays,), jnp.float32),
        scratch_types=[
            pltpu.VMEM((chunk,), jnp.int32),
            pltpu.VMEM((3 * chunk,), jnp.int32),
            pltpu.VMEM((chunk,), jnp.float32),
            pltpu.VMEM((3 * chunk,), jnp.float32),
            pltpu.VMEM((rpw,), jnp.int32),
            pltpu.VMEM((rpw,), jnp.float32),
            pltpu.SemaphoreType.DMA,
        ],
    )
    def stage_b(ch, cuh, oh, cu_v, idx_v, m_v, g_v, oidx_v, o_v, sem):
        wid = lax.axis_index("s") * 2 + lax.axis_index("c")
        base = wid * rpw
        pltpu.sync_copy(cuh.at[pl.ds(base, chunk)], cu_v)
        for q in range(0, chunk, 16):
            cu16 = cu_v[pl.ds(q, 16)]
            flat = jnp.maximum(cu16 - 1, 0) * 3
            for c in range(3):
                idx_v[pl.ds(c * chunk + q, 16)] = flat + c
            m_v[pl.ds(q, 16)] = jnp.where(cu16 > 0, 1.0, 0.0)
        pltpu.async_copy(ch.at[idx_v], g_v, sem).wait()
        lane = lax.iota(jnp.int32, 16)
        for c in range(3):
            for q in range(0, rpw, 16):
                m_lo = m_v[pl.ds(q, 16)]
                m_hi = m_v[pl.ds(q + 1, 16)]
                glo = g_v[pl.ds(c * chunk + q, 16)] * m_lo
                ghi = g_v[pl.ds(c * chunk + q + 1, 16)] * m_hi
                o_v[pl.ds(q, 16)] = ghi - glo
                oidx_v[pl.ds(q, 16)] = (base + q + lane) * 3 + c
            pltpu.async_copy(o_v, oh.at[oidx_v], sem).wait()

    return stage_b


def kernel(value_samples, weights_samples, cu_seqlens):
    total = value_samples.shape[0]
    n_rays = cu_seqlens.shape[0] - 1
    rows = total // LANES

    v_int = value_samples.reshape(rows, ILANES)
    w2 = weights_samples.reshape(rows, LANES)
    c_int = _stage_a(v_int, w2)

    rpw = n_rays // NW
    chunk = rpw + 16
    pad = NW * rpw + chunk - (n_rays + 1)
    cu_pad = jnp.concatenate(
        [cu_seqlens.astype(jnp.int32),
         jnp.full((pad,), total, dtype=jnp.int32)])

    stage_b = _make_stage_b(n_rays, chunk)
    out_flat = stage_b(c_int.reshape(3 * total), cu_pad)
    return out_flat.reshape(n_rays, 3)


# speedup vs baseline: 1.4710x; 1.4710x over previous
import functools

import numpy as np
import jax
import jax.numpy as jnp
from jax import lax
from jax.experimental import pallas as pl
from jax.experimental.pallas import tpu as pltpu
from jax.experimental.pallas import tpu_sc as plsc

LANES = 128
ILANES = 3 * LANES
BR = 256
NW = 32

_K = np.arange(LANES)
_J = np.arange(ILANES)
_E = (_J[None, :] // 3 == _K[:, None]).astype(np.float32)
_U3 = ((_J[:, None] <= _J[None, :]) &
       (_J[:, None] % 3 == _J[None, :] % 3)).astype(np.float32)
_B = np.arange(BR)
_LS = (_B[None, :] < _B[:, None]).astype(np.float32)
_R3 = (_J[None, :] % 3 == np.arange(3)[:, None]).astype(np.float32)

_HI = lax.Precision.HIGHEST


def _dot(a, b):
    return jnp.dot(a, b, preferred_element_type=jnp.float32, precision=_HI)


def _cumsum_body(v_ref, w_ref, e_ref, u3_ref, ls_ref, r3_ref, out_ref,
                 carry_ref):
    b = pl.program_id(0)

    @pl.when(b == 0)
    def _init():
        carry_ref[...] = jnp.zeros((1, 3), jnp.float32)

    w3 = _dot(w_ref[...], e_ref[...])
    x3 = v_ref[...] * w3
    incl = _dot(x3, u3_ref[...])
    srow3 = incl[:, ILANES - 3:ILANES]
    rowcarry = _dot(ls_ref[...], srow3)
    rc3 = rowcarry + carry_ref[...]
    out_ref[...] = incl + _dot(rc3, r3_ref[...])
    carry_ref[...] = carry_ref[...] + (rowcarry + srow3)[BR - 1:BR, :]


def _stage_a(v_int, w2):
    rows = v_int.shape[0]
    nblocks = rows // BR
    const_spec = lambda shape: pl.BlockSpec(shape, lambda b: (0, 0))
    return pl.pallas_call(
        _cumsum_body,
        grid=(nblocks,),
        in_specs=[
            pl.BlockSpec((BR, ILANES), lambda b: (b, 0)),
            pl.BlockSpec((BR, LANES), lambda b: (b, 0)),
            const_spec((LANES, ILANES)),
            const_spec((ILANES, ILANES)),
            const_spec((BR, BR)),
            const_spec((3, ILANES)),
        ],
        out_specs=pl.BlockSpec((BR, ILANES), lambda b: (b, 0)),
        out_shape=jax.ShapeDtypeStruct((rows, ILANES), jnp.float32),
        scratch_shapes=[pltpu.VMEM((1, 3), jnp.float32)],
    )(v_int, w2, jnp.asarray(_E), jnp.asarray(_U3), jnp.asarray(_LS),
      jnp.asarray(_R3))


def _make_stage_b(n_rays, chunk):
    rpw = n_rays // NW
    mesh = plsc.VectorSubcoreMesh(core_axis_name="c", subcore_axis_name="s")

    @functools.partial(
        pl.kernel,
        mesh=mesh,
        out_type=[jax.ShapeDtypeStruct((n_rays,), jnp.float32)] * 3,
        scratch_types=[
            pltpu.VMEM((chunk,), jnp.int32),
            pltpu.VMEM((3 * chunk,), jnp.int32),
            pltpu.VMEM((chunk,), jnp.float32),
            pltpu.VMEM((3 * chunk,), jnp.float32),
            pltpu.VMEM((rpw,), jnp.float32),
            pltpu.SemaphoreType.DMA,
        ],
    )
    def stage_b(ch, cuh, o0h, o1h, o2h, cu_v, idx_v, m_v, g_v, o_v, sem):
        wid = lax.axis_index("s") * 2 + lax.axis_index("c")
        base = wid * rpw
        pltpu.sync_copy(cuh.at[pl.ds(base, chunk)], cu_v)
        for q in range(0, chunk, 16):
            cu16 = cu_v[pl.ds(q, 16)]
            flat = jnp.maximum(cu16 - 1, 0) * 3
            for c in range(3):
                idx_v[pl.ds(c * chunk + q, 16)] = flat + c
            m_v[pl.ds(q, 16)] = jnp.where(cu16 > 0, 1.0, 0.0)
        pltpu.async_copy(ch.at[idx_v], g_v, sem).wait()
        for c, oh in enumerate((o0h, o1h, o2h)):
            for q in range(0, rpw, 16):
                m_lo = m_v[pl.ds(q, 16)]
                m_hi = m_v[pl.ds(q + 1, 16)]
                glo = g_v[pl.ds(c * chunk + q, 16)] * m_lo
                ghi = g_v[pl.ds(c * chunk + q + 1, 16)] * m_hi
                o_v[pl.ds(q, 16)] = ghi - glo
            pltpu.sync_copy(o_v, oh.at[pl.ds(base, rpw)])

    return stage_b


def kernel(value_samples, weights_samples, cu_seqlens):
    total = value_samples.shape[0]
    n_rays = cu_seqlens.shape[0] - 1
    rows = total // LANES

    v_int = value_samples.reshape(rows, ILANES)
    w2 = weights_samples.reshape(rows, LANES)
    c_int = _stage_a(v_int, w2)

    rpw = n_rays // NW
    chunk = rpw + 16
    pad = NW * rpw + chunk - (n_rays + 1)
    cu_pad = jnp.concatenate(
        [cu_seqlens.astype(jnp.int32),
         jnp.full((pad,), total, dtype=jnp.int32)])

    stage_b = _make_stage_b(n_rays, chunk)
    o0, o1, o2 = stage_b(c_int.reshape(3 * total), cu_pad)
    return jnp.stack([o0, o1, o2], axis=1)


# speedup vs baseline: 7.2567x; 4.9332x over previous
import functools

import jax
import jax.numpy as jnp
from jax import lax
from jax.experimental import pallas as pl
from jax.experimental.pallas import tpu as pltpu
from jax.experimental.pallas import tpu_sc as plsc

LANES = 128
BR = 256
NW = 32


def _cumsum_body(v_ref, w_ref, out_ref, carry_ref):
    b = pl.program_id(0)

    @pl.when(b == 0)
    def _init():
        for c in range(3):
            carry_ref[c] = 0.0

    ii = lax.broadcasted_iota(jnp.int32, (LANES, LANES), 0)
    jj = lax.broadcasted_iota(jnp.int32, (LANES, LANES), 1)
    upper_incl = (ii <= jj).astype(jnp.float32)
    aa = lax.broadcasted_iota(jnp.int32, (BR, BR), 0)
    bb = lax.broadcasted_iota(jnp.int32, (BR, BR), 1)
    strict_lower = (bb < aa).astype(jnp.float32)

    w = w_ref[...]
    for c in range(3):
        x = v_ref[c] * w
        xh = x.astype(jnp.bfloat16).astype(jnp.float32)
        xl = x - xh
        incl = (jnp.dot(xh, upper_incl, preferred_element_type=jnp.float32)
                + jnp.dot(xl, upper_incl, preferred_element_type=jnp.float32))
        row_sums = incl[:, LANES - 1:LANES]
        row_carry = jnp.dot(strict_lower, row_sums,
                            preferred_element_type=jnp.float32,
                            precision=lax.Precision.HIGHEST)
        out_ref[c] = incl + row_carry + carry_ref[c]
        carry_ref[c] = carry_ref[c] + jnp.sum(row_sums)


def _stage_a(v3, w2):
    rows = v3.shape[1]
    nblocks = rows // BR
    return pl.pallas_call(
        _cumsum_body,
        grid=(nblocks,),
        in_specs=[
            pl.BlockSpec((3, BR, LANES), lambda b: (0, b, 0)),
            pl.BlockSpec((BR, LANES), lambda b: (b, 0)),
        ],
        out_specs=pl.BlockSpec((3, BR, LANES), lambda b: (0, b, 0)),
        out_shape=jax.ShapeDtypeStruct((3, rows, LANES), jnp.float32),
        scratch_shapes=[pltpu.SMEM((3,), jnp.float32)],
    )(v3, w2)


def _make_stage_b(n_rays, total, chunk):
    rpw = n_rays // NW
    mesh = plsc.VectorSubcoreMesh(core_axis_name="c", subcore_axis_name="s")

    @functools.partial(
        pl.kernel,
        mesh=mesh,
        out_type=[jax.ShapeDtypeStruct((n_rays,), jnp.float32)] * 3,
        scratch_types=[
            pltpu.VMEM((chunk,), jnp.int32),
            pltpu.VMEM((3 * chunk,), jnp.int32),
            pltpu.VMEM((chunk,), jnp.float32),
            pltpu.VMEM((3 * chunk,), jnp.float32),
            pltpu.VMEM((rpw,), jnp.float32),
            pltpu.SemaphoreType.DMA,
        ],
    )
    def stage_b(ch, cuh, o0h, o1h, o2h, cu_v, idx_v, m_v, g_v, o_v, sem):
        wid = lax.axis_index("s") * 2 + lax.axis_index("c")
        base = wid * rpw
        pltpu.sync_copy(cuh.at[pl.ds(base, chunk)], cu_v)
        for q in range(0, chunk, 16):
            cu16 = cu_v[pl.ds(q, 16)]
            flat = jnp.maximum(cu16 - 1, 0)
            for c in range(3):
                idx_v[pl.ds(c * chunk + q, 16)] = flat + c * total
            m_v[pl.ds(q, 16)] = jnp.where(cu16 > 0, 1.0, 0.0)
        pltpu.async_copy(ch.at[idx_v], g_v, sem).wait()
        for c, oh in enumerate((o0h, o1h, o2h)):
            for q in range(0, rpw, 16):
                m_lo = m_v[pl.ds(q, 16)]
                m_hi = m_v[pl.ds(q + 1, 16)]
                glo = g_v[pl.ds(c * chunk + q, 16)] * m_lo
                ghi = g_v[pl.ds(c * chunk + q + 1, 16)] * m_hi
                o_v[pl.ds(q, 16)] = ghi - glo
            pltpu.sync_copy(o_v, oh.at[pl.ds(base, rpw)])

    return stage_b


def kernel(value_samples, weights_samples, cu_seqlens):
    total = value_samples.shape[0]
    n_rays = cu_seqlens.shape[0] - 1
    rows = total // LANES

    v3 = value_samples.T.reshape(3, rows, LANES)
    w2 = weights_samples.reshape(rows, LANES)
    c3 = _stage_a(v3, w2)

    rpw = n_rays // NW
    chunk = rpw + 16
    pad = NW * rpw + chunk - (n_rays + 1)
    cu_pad = jnp.concatenate(
        [cu_seqlens.astype(jnp.int32),
         jnp.full((pad,), total, dtype=jnp.int32)])

    stage_b = _make_stage_b(n_rays, total, chunk)
    o0, o1, o2 = stage_b(c3.reshape(3 * total), cu_pad)
    return jnp.stack([o0, o1, o2], axis=1)


# speedup vs baseline: 7.4101x; 1.0211x over previous
import functools

import jax
import jax.numpy as jnp
from jax import lax
from jax.experimental import pallas as pl
from jax.experimental.pallas import tpu as pltpu
from jax.experimental.pallas import tpu_sc as plsc

LANES = 128
BR = 256
NW = 32


def _cumsum_body(v_ref, w_ref, out_ref, carry_ref):
    b = pl.program_id(0)

    @pl.when(b == 0)
    def _init():
        for c in range(3):
            carry_ref[c] = 0.0

    ii = lax.broadcasted_iota(jnp.int32, (LANES, LANES), 0)
    jj = lax.broadcasted_iota(jnp.int32, (LANES, LANES), 1)
    upper_incl = (ii <= jj).astype(jnp.float32)
    aa = lax.broadcasted_iota(jnp.int32, (BR, BR), 0)
    bb = lax.broadcasted_iota(jnp.int32, (BR, BR), 1)
    strict_lower = (bb < aa).astype(jnp.float32)

    x_all = (v_ref[...] * w_ref[...]).reshape(3 * BR, LANES)
    xh = x_all.astype(jnp.bfloat16).astype(jnp.float32)
    xl = x_all - xh
    incl = (jnp.dot(xh, upper_incl, preferred_element_type=jnp.float32)
            + jnp.dot(xl, upper_incl, preferred_element_type=jnp.float32))
    s_all = incl[:, LANES - 1:LANES]
    for c in range(3):
        row_sums = s_all[c * BR:(c + 1) * BR]
        row_carry = jnp.dot(strict_lower, row_sums,
                            preferred_element_type=jnp.float32,
                            precision=lax.Precision.HIGHEST)
        out_ref[c] = incl[c * BR:(c + 1) * BR] + row_carry + carry_ref[c]
        carry_ref[c] = carry_ref[c] + jnp.sum(row_sums)


def _stage_a(v3, w2):
    rows = v3.shape[1]
    nblocks = rows // BR
    return pl.pallas_call(
        _cumsum_body,
        grid=(nblocks,),
        in_specs=[
            pl.BlockSpec((3, BR, LANES), lambda b: (0, b, 0)),
            pl.BlockSpec((BR, LANES), lambda b: (b, 0)),
        ],
        out_specs=pl.BlockSpec((3, BR, LANES), lambda b: (0, b, 0)),
        out_shape=jax.ShapeDtypeStruct((3, rows, LANES), jnp.float32),
        scratch_shapes=[pltpu.SMEM((3,), jnp.float32)],
    )(v3, w2)


def _make_stage_b(n_rays, total, chunk):
    rpw = n_rays // NW
    mesh = plsc.VectorSubcoreMesh(core_axis_name="c", subcore_axis_name="s")

    @functools.partial(
        pl.kernel,
        mesh=mesh,
        out_type=[jax.ShapeDtypeStruct((n_rays,), jnp.float32)] * 3,
        scratch_types=[
            pltpu.VMEM((chunk,), jnp.int32),
            pltpu.VMEM((3 * chunk,), jnp.int32),
            pltpu.VMEM((chunk,), jnp.float32),
            pltpu.VMEM((3 * chunk,), jnp.float32),
            pltpu.VMEM((rpw,), jnp.float32),
            pltpu.SemaphoreType.DMA,
        ],
    )
    def stage_b(ch, cuh, o0h, o1h, o2h, cu_v, idx_v, m_v, g_v, o_v, sem):
        wid = lax.axis_index("s") * 2 + lax.axis_index("c")
        base = wid * rpw
        pltpu.sync_copy(cuh.at[pl.ds(base, chunk)], cu_v)
        for q in range(0, chunk, 16):
            cu16 = cu_v[pl.ds(q, 16)]
            flat = jnp.maximum(cu16 - 1, 0)
            for c in range(3):
                idx_v[pl.ds(c * chunk + q, 16)] = flat + c * total
            m_v[pl.ds(q, 16)] = jnp.where(cu16 > 0, 1.0, 0.0)
        pltpu.async_copy(ch.at[idx_v], g_v, sem).wait()
        for c, oh in enumerate((o0h, o1h, o2h)):
            for q in range(0, rpw, 16):
                m_lo = m_v[pl.ds(q, 16)]
                m_hi = m_v[pl.ds(q + 1, 16)]
                glo = g_v[pl.ds(c * chunk + q, 16)] * m_lo
                ghi = g_v[pl.ds(c * chunk + q + 1, 16)] * m_hi
                o_v[pl.ds(q, 16)] = ghi - glo
            pltpu.sync_copy(o_v, oh.at[pl.ds(base, rpw)])

    return stage_b


def kernel(value_samples, weights_samples, cu_seqlens):
    total = value_samples.shape[0]
    n_rays = cu_seqlens.shape[0] - 1
    rows = total // LANES

    v3 = value_samples.T.reshape(3, rows, LANES)
    w2 = weights_samples.reshape(rows, LANES)
    c3 = _stage_a(v3, w2)

    rpw = n_rays // NW
    chunk = rpw + 16
    pad = NW * rpw + chunk - (n_rays + 1)
    cu_pad = jnp.concatenate(
        [cu_seqlens.astype(jnp.int32),
         jnp.full((pad,), total, dtype=jnp.int32)])

    stage_b = _make_stage_b(n_rays, total, chunk)
    o0, o1, o2 = stage_b(c3.reshape(3 * total), cu_pad)
    return jnp.stack([o0, o1, o2], axis=1)


# speedup vs baseline: 7.7786x; 1.0497x over previous
import functools

import jax
import jax.numpy as jnp
from jax import lax
from jax.experimental import pallas as pl
from jax.experimental.pallas import tpu as pltpu
from jax.experimental.pallas import tpu_sc as plsc

LANES = 128
BR = 256
NW = 32


def _cumsum_body(v_ref, w_ref, out_ref, carry_ref):
    b = pl.program_id(0)

    @pl.when(b == 0)
    def _init():
        for c in range(3):
            carry_ref[c] = 0.0

    ii = lax.broadcasted_iota(jnp.int32, (LANES, LANES), 0)
    jj = lax.broadcasted_iota(jnp.int32, (LANES, LANES), 1)
    upper_incl = (ii <= jj).astype(jnp.float32)
    aa = lax.broadcasted_iota(jnp.int32, (BR, BR), 0)
    bb = lax.broadcasted_iota(jnp.int32, (BR, BR), 1)
    strict_lower = (bb < aa).astype(jnp.float32)

    x_all = (v_ref[...] * w_ref[...]).reshape(3 * BR, LANES)
    xh = x_all.astype(jnp.bfloat16).astype(jnp.float32)
    xl = x_all - xh
    incl = (jnp.dot(xh, upper_incl, preferred_element_type=jnp.float32)
            + jnp.dot(xl, upper_incl, preferred_element_type=jnp.float32))
    s_all = incl[:, LANES - 1:LANES]
    s3 = jnp.concatenate(
        [s_all[c * BR:(c + 1) * BR] for c in range(3)], axis=1)
    rc3 = jnp.dot(strict_lower, s3, preferred_element_type=jnp.float32,
                  precision=lax.Precision.HIGHEST)
    for c in range(3):
        out_ref[c] = incl[c * BR:(c + 1) * BR] + rc3[:, c:c + 1] + carry_ref[c]
        carry_ref[c] = carry_ref[c] + jnp.sum(s3[:, c:c + 1])


def _stage_a(v3, w2):
    rows = v3.shape[1]
    nblocks = rows // BR
    return pl.pallas_call(
        _cumsum_body,
        grid=(nblocks,),
        in_specs=[
            pl.BlockSpec((3, BR, LANES), lambda b: (0, b, 0)),
            pl.BlockSpec((BR, LANES), lambda b: (b, 0)),
        ],
        out_specs=pl.BlockSpec((3, BR, LANES), lambda b: (0, b, 0)),
        out_shape=jax.ShapeDtypeStruct((3, rows, LANES), jnp.float32),
        scratch_shapes=[pltpu.SMEM((3,), jnp.float32)],
    )(v3, w2)


def _make_stage_b(n_rays, total, chunk):
    rpw = n_rays // NW
    mesh = plsc.VectorSubcoreMesh(core_axis_name="c", subcore_axis_name="s")

    @functools.partial(
        pl.kernel,
        mesh=mesh,
        out_type=[jax.ShapeDtypeStruct((n_rays,), jnp.float32)] * 3,
        scratch_types=[
            pltpu.VMEM((chunk,), jnp.int32),
            pltpu.VMEM((3 * chunk,), jnp.int32),
            pltpu.VMEM((chunk,), jnp.float32),
            pltpu.VMEM((3 * chunk,), jnp.float32),
            pltpu.VMEM((rpw,), jnp.float32),
            pltpu.SemaphoreType.DMA,
        ],
    )
    def stage_b(ch, cuh, o0h, o1h, o2h, cu_v, idx_v, m_v, g_v, o_v, sem):
        wid = lax.axis_index("s") * 2 + lax.axis_index("c")
        base = wid * rpw
        pltpu.sync_copy(cuh.at[pl.ds(base, chunk)], cu_v)
        for q in range(0, chunk, 16):
            cu16 = cu_v[pl.ds(q, 16)]
            flat = jnp.maximum(cu16 - 1, 0)
            for c in range(3):
                idx_v[pl.ds(c * chunk + q, 16)] = flat + c * total
            m_v[pl.ds(q, 16)] = jnp.where(cu16 > 0, 1.0, 0.0)
        pltpu.async_copy(ch.at[idx_v], g_v, sem).wait()
        for c, oh in enumerate((o0h, o1h, o2h)):
            for q in range(0, rpw, 16):
                m_lo = m_v[pl.ds(q, 16)]
                m_hi = m_v[pl.ds(q + 1, 16)]
                glo = g_v[pl.ds(c * chunk + q, 16)] * m_lo
                ghi = g_v[pl.ds(c * chunk + q + 1, 16)] * m_hi
                o_v[pl.ds(q, 16)] = ghi - glo
            pltpu.sync_copy(o_v, oh.at[pl.ds(base, rpw)])

    return stage_b


def kernel(value_samples, weights_samples, cu_seqlens):
    total = value_samples.shape[0]
    n_rays = cu_seqlens.shape[0] - 1
    rows = total // LANES

    v3 = value_samples.T.reshape(3, rows, LANES)
    w2 = weights_samples.reshape(rows, LANES)
    c3 = _stage_a(v3, w2)

    rpw = n_rays // NW
    chunk = rpw + 16
    pad = NW * rpw + chunk - (n_rays + 1)
    cu_pad = jnp.concatenate(
        [cu_seqlens.astype(jnp.int32),
         jnp.full((pad,), total, dtype=jnp.int32)])

    stage_b = _make_stage_b(n_rays, total, chunk)
    o0, o1, o2 = stage_b(c3.reshape(3 * total), cu_pad)
    return jnp.stack([o0, o1, o2], axis=1)
